# baseline (speedup 1.0000x reference)
import functools

import jax
import jax.numpy as jnp
from jax import lax
from jax.experimental import pallas as pl
from jax.experimental.pallas import tpu as pltpu
from jax.experimental.pallas import tpu_sc as plsc

NUM_CORES = 2
NUM_SUBCORES = 16
NW = NUM_CORES * NUM_SUBCORES


def _make_sc_gather(b_rows: int, num_fields: int, emb_dim: int):
    chunk = b_rows // NW
    mesh = plsc.VectorSubcoreMesh(core_axis_name="c", subcore_axis_name="s")

    @functools.partial(
        pl.kernel,
        out_type=jax.ShapeDtypeStruct((b_rows, num_fields * emb_dim),
                                      jnp.float32),
        mesh=mesh,
        scratch_types=[
            pltpu.VMEM((chunk,), jnp.int32),
            pltpu.VMEM((chunk, emb_dim), jnp.float32),
            pltpu.SemaphoreType.DMA,
        ],
        compiler_params=pltpu.CompilerParams(use_tc_tiling_on_sc=False),
    )
    def sc_gather(tables_hbm, xcat_hbm, out_hbm, idx_v, rows_v, sem):
        wid = lax.axis_index("s") * NUM_CORES + lax.axis_index("c")
        base = wid * chunk

        def field_body(i, carry):
            pltpu.sync_copy(xcat_hbm.at[i, pl.ds(base, chunk)], idx_v)
            pltpu.async_copy(tables_hbm.at[i].at[idx_v], rows_v, sem).wait()
            pltpu.sync_copy(
                rows_v,
                out_hbm.at[pl.ds(base, chunk), pl.ds(i * emb_dim, emb_dim)])
            return carry

        lax.fori_loop(0, num_fields, field_body, 0)

    return sc_gather


def _mlp_body(cat_ref, num_ref, w1c_ref, w1n_ref, b1_ref, w2_ref, b2_ref,
              out_ref):
    h = jnp.dot(cat_ref[...], w1c_ref[...], preferred_element_type=jnp.float32)
    h = h + jnp.dot(num_ref[...], w1n_ref[...],
                    preferred_element_type=jnp.float32)
    h = jnp.maximum(h + b1_ref[...], 0.0)
    out_ref[...] = (
        jnp.dot(h, w2_ref[...], preferred_element_type=jnp.float32)
        + b2_ref[...]
    )


def _tc_mlp(cat_emb, x_num, w1c, w1n, b1, w2, b2, blk: int):
    b_rows = cat_emb.shape[0]
    grid = (b_rows // blk,)
    return pl.pallas_call(
        _mlp_body,
        grid=grid,
        in_specs=[
            pl.BlockSpec((blk, cat_emb.shape[1]), lambda i: (i, 0)),
            pl.BlockSpec((blk, x_num.shape[1]), lambda i: (i, 0)),
            pl.BlockSpec(w1c.shape, lambda i: (0, 0)),
            pl.BlockSpec(w1n.shape, lambda i: (0, 0)),
            pl.BlockSpec(b1.shape, lambda i: (0, 0)),
            pl.BlockSpec(w2.shape, lambda i: (0, 0)),
            pl.BlockSpec(b2.shape, lambda i: (0, 0)),
        ],
        out_specs=pl.BlockSpec((blk, 1), lambda i: (i, 0)),
        out_shape=jax.ShapeDtypeStruct((b_rows, 1), jnp.float32),
    )(cat_emb, x_num, w1c, w1n, b1, w2, b2)


def kernel(X_cat, X_num, tables, W1, b1, W2, b2):
    b_rows, num_fields = X_cat.shape
    emb_dim = tables.shape[2]

    xcat_t = X_cat.astype(jnp.int32).T
    cat_emb = _make_sc_gather(b_rows, num_fields, emb_dim)(tables, xcat_t)

    w1c = W1[:, : num_fields * emb_dim].T
    w1n = W1[:, num_fields * emb_dim:].T
    out = _tc_mlp(cat_emb, X_num, w1c, w1n, b1[None, :], W2.T,
                  b2[None, :], blk=2048)
    return out[:, 0]


# speedup vs baseline: 1.0049x; 1.0049x over previous
import functools

import jax
import jax.numpy as jnp
from jax import lax
from jax.experimental import pallas as pl
from jax.experimental.pallas import tpu as pltpu
from jax.experimental.pallas import tpu_sc as plsc

NUM_CORES = 2
NUM_SUBCORES = 16
NW = NUM_CORES * NUM_SUBCORES


def _make_sc_gather(b_rows: int, num_fields: int, emb_dim: int):
    chunk = b_rows // NW
    mesh = plsc.VectorSubcoreMesh(core_axis_name="c", subcore_axis_name="s")

    @functools.partial(
        pl.kernel,
        out_type=jax.ShapeDtypeStruct((b_rows, num_fields * emb_dim),
                                      jnp.float32),
        mesh=mesh,
        scratch_types=[
            pltpu.VMEM((chunk, num_fields), jnp.int32),
            pltpu.VMEM((chunk,), jnp.int32),
            pltpu.VMEM((chunk, emb_dim), jnp.float32),
            pltpu.SemaphoreType.DMA,
        ],
        compiler_params=pltpu.CompilerParams(use_tc_tiling_on_sc=False,
                                             needs_layout_passes=False),
    )
    def sc_gather(tables_hbm, xcat_hbm, out_hbm, xcat_v, idx_v, rows_v, sem):
        wid = lax.axis_index("s") * NUM_CORES + lax.axis_index("c")
        base = wid * chunk

        pltpu.sync_copy(xcat_hbm.at[pl.ds(base, chunk)], xcat_v)
        lane = lax.iota(jnp.int32, 16)

        def field_body(i, carry):
            col = jnp.full((16,), i, jnp.int32)

            def extract_body(t, carry2):
                rows16 = lane + t * 16
                vals = plsc.load_gather(xcat_v, [rows16, col])
                idx_v[pl.ds(t * 16, 16)] = vals
                return carry2

            lax.fori_loop(0, chunk // 16, extract_body, 0)
            pltpu.async_copy(tables_hbm.at[i].at[idx_v], rows_v, sem).wait()
            pltpu.sync_copy(
                rows_v,
                out_hbm.at[pl.ds(base, chunk), pl.ds(i * emb_dim, emb_dim)])
            return carry

        lax.fori_loop(0, num_fields, field_body, 0)

    return sc_gather


def _mlp_body(cat_ref, num_ref, w1c_ref, w1n_ref, b1_ref, w2_ref, b2_ref,
              out_ref):
    h = jnp.dot(cat_ref[...], w1c_ref[...], preferred_element_type=jnp.float32)
    h = h + jnp.dot(num_ref[...], w1n_ref[...],
                    preferred_element_type=jnp.float32)
    h = jnp.maximum(h + b1_ref[...], 0.0)
    out_ref[...] = (
        jnp.dot(h, w2_ref[...], preferred_element_type=jnp.float32)
        + b2_ref[...]
    )


def _tc_mlp(cat_emb, x_num, w1c, w1n, b1, w2, b2, blk: int):
    b_rows = cat_emb.shape[0]
    grid = (b_rows // blk,)
    return pl.pallas_call(
        _mlp_body,
        grid=grid,
        in_specs=[
            pl.BlockSpec((blk, cat_emb.shape[1]), lambda i: (i, 0)),
            pl.BlockSpec((blk, x_num.shape[1]), lambda i: (i, 0)),
            pl.BlockSpec(w1c.shape, lambda i: (0, 0)),
            pl.BlockSpec(w1n.shape, lambda i: (0, 0)),
            pl.BlockSpec(b1.shape, lambda i: (0, 0)),
            pl.BlockSpec(w2.shape, lambda i: (0, 0)),
            pl.BlockSpec(b2.shape, lambda i: (0, 0)),
        ],
        out_specs=pl.BlockSpec((blk, 1), lambda i: (i, 0)),
        out_shape=jax.ShapeDtypeStruct((b_rows, 1), jnp.float32),
    )(cat_emb, x_num, w1c, w1n, b1, w2, b2)


def kernel(X_cat, X_num, tables, W1, b1, W2, b2):
    b_rows, num_fields = X_cat.shape
    emb_dim = tables.shape[2]

    cat_emb = _make_sc_gather(b_rows, num_fields, emb_dim)(tables, X_cat)

    w1c = W1[:, : num_fields * emb_dim].T
    w1n = W1[:, num_fields * emb_dim:].T
    out = _tc_mlp(cat_emb, X_num, w1c, w1n, b1[None, :], W2.T,
                  b2[None, :], blk=2048)
    return out[:, 0]
